# baseline (speedup 1.0000x reference)
import functools

import jax
import jax.numpy as jnp
from jax import lax
from jax.experimental import pallas as pl
from jax.experimental.pallas import tpu as pltpu
from jax.experimental.pallas import tpu_sc as plsc

NC = 2
NS = 16
LANES = 16
NW = NC * NS
CP = 128
CLS_PER_W = CP // NW


@functools.partial(jax.jit, static_argnames=("A", "A_P"))
def _sc_pool(logits_t, idx_flat, w_flat, *, A, A_P):
    V, B = logits_t.shape
    n_ent = CP * A_P
    vec_b = B // LANES

    mesh = plsc.VectorSubcoreMesh(
        core_axis_name="c", subcore_axis_name="s", num_cores=NC, num_subcores=NS
    )

    @functools.partial(
        pl.kernel,
        mesh=mesh,
        compiler_params=pltpu.CompilerParams(needs_layout_passes=False),
        out_type=jax.ShapeDtypeStruct((CP, B), jnp.float32),
        scratch_types=[
            pltpu.VMEM((CP, A_P), jnp.int32),
            pltpu.VMEM((n_ent,), jnp.float32),
            pltpu.VMEM((A * LANES,), jnp.float32),
            pltpu.VMEM((A_P, B), jnp.float32),
            pltpu.VMEM((B,), jnp.float32),
            pltpu.SemaphoreType.DMA,
        ],
    )
    def pool(lt_hbm, idx_hbm, w_hbm, out_hbm, idx_v, w_v, ws_v, rows_v, acc_v,
             sem):
        wid = lax.axis_index("s") * NC + lax.axis_index("c")
        pltpu.sync_copy(idx_hbm, idx_v)
        pltpu.sync_copy(w_hbm, w_v)

        def cls_body(k, carry):
            c = wid * CLS_PER_W + k
            ent0 = c * A_P
            pltpu.async_copy(lt_hbm.at[idx_v.at[c]], rows_v, sem).wait()

            def splat(a, c2):
                ws_v[pl.ds(a * LANES, LANES)] = plsc.load_gather(
                    w_v, [jnp.full((LANES,), ent0 + a, jnp.int32)]
                )
                return c2

            lax.fori_loop(0, A, splat, 0)

            def col_body(q, c2):
                sl = pl.ds(q * LANES, LANES)

                def a_body(a, acc):
                    return acc + rows_v[a, sl] * ws_v[pl.ds(a * LANES, LANES)]

                acc_v[sl] = lax.fori_loop(
                    0, A, a_body, jnp.zeros((LANES,), jnp.float32)
                )
                return c2

            lax.fori_loop(0, vec_b, col_body, 0)
            pltpu.sync_copy(acc_v, out_hbm.at[c])
            return carry

        lax.fori_loop(0, CLS_PER_W, cls_body, 0)

    return pool(logits_t, idx_flat, w_flat)


def _lse_body(x_ref, out_ref, m_ref, sum_ref, *, nv, vb):
    j = pl.program_id(0)
    i = pl.program_id(1)

    LOG2E = 1.4426950408889634
    x = x_ref[...]
    bm = jnp.max(x, axis=0, keepdims=True)
    ones = jnp.ones((1, vb), jnp.float32)

    @pl.when(j == 0)
    def _init():
        m_ref[i] = bm
        e = jnp.exp2(x * LOG2E - bm * LOG2E)
        sum_ref[i] = jnp.dot(ones, e, preferred_element_type=jnp.float32)

    @pl.when(j > 0)
    def _accum():
        m_old = m_ref[i]
        m_new = jnp.maximum(m_old, bm)
        alpha = jnp.where(m_old == m_new, 1.0, jnp.exp(m_old - m_new))
        e = jnp.exp2(x * LOG2E - m_new * LOG2E)
        es = jnp.dot(ones, e, preferred_element_type=jnp.float32)
        sum_ref[i] = sum_ref[i] * alpha + es
        m_ref[i] = m_new

    @pl.when(j == nv - 1)
    def _finalize():
        out_ref[...] = m_ref[i] + jnp.log(sum_ref[i])


@functools.partial(jax.jit, static_argnames=("Bb", "vb"))
def _lse(logits_t, *, Bb, vb):
    V, B = logits_t.shape
    nb = B // Bb
    nv = V // vb

    return pl.pallas_call(
        functools.partial(_lse_body, nv=nv, vb=vb),
        grid=(nv, nb),
        in_specs=[pl.BlockSpec((vb, Bb), lambda j, i: (j, i))],
        out_specs=pl.BlockSpec((1, Bb), lambda j, i: (0, i)),
        out_shape=jax.ShapeDtypeStruct((1, B), jnp.float32),
        scratch_shapes=[
            pltpu.VMEM((nb, 1, Bb), jnp.float32),
            pltpu.VMEM((nb, 1, Bb), jnp.float32),
        ],
        compiler_params=pltpu.CompilerParams(
            dimension_semantics=("arbitrary", "arbitrary"),
        ),
    )(logits_t)


def _combine_body(pooled_ref, norm_ref, mask_ref, out_ref, *, C):
    mask = mask_ref[...]
    summask = jnp.sum(mask, axis=1, keepdims=True)
    denom = jnp.clip(summask, 1e-9, None)
    inv = 1.0 / denom
    scale = summask * inv
    res = pooled_ref[...] * inv - scale * norm_ref[...]
    out_ref[...] = res[:C, :]


@functools.partial(jax.jit, static_argnames=("C", "Bb"))
def _combine(pooled, norm, mask_cp, *, C, Bb):
    _, B = pooled.shape
    nb = B // Bb

    return pl.pallas_call(
        functools.partial(_combine_body, C=C),
        grid=(nb,),
        in_specs=[
            pl.BlockSpec((CP, Bb), lambda i: (0, i)),
            pl.BlockSpec((1, Bb), lambda i: (0, i)),
            pl.BlockSpec(mask_cp.shape, lambda i: (0, 0)),
        ],
        out_specs=pl.BlockSpec((C, Bb), lambda i: (0, i)),
        out_shape=jax.ShapeDtypeStruct((C, B), jnp.float32),
    )(pooled, norm, mask_cp)


def kernel(logits, word2label, label_words_mask):
    B, V = logits.shape
    C, A = word2label.shape
    assert C <= CP and B % (LANES * NW) == 0

    A_P = -(-A // 8) * 8
    idx_cp = jnp.zeros((CP, A_P), jnp.int32).at[:C, :A].set(word2label)
    w_cp = jnp.zeros((CP, A_P), jnp.float32).at[:C, :A].set(
        label_words_mask.astype(jnp.float32)
    )
    mask_cp = w_cp[:, :A]

    lt = logits.T
    pooled = _sc_pool(lt, idx_cp, w_cp.reshape(-1), A=A, A_P=A_P)

    vb = next(d for d in (5000, 4000, 2000, 1000, 800, 500, 400, 250, 200, 100, 50,
                          25, 20, 10, 8, 5, 4, 2, 1) if V % d == 0)
    norm = _lse(lt, Bb=B, vb=vb)

    out_t = _combine(pooled, norm, mask_cp, C=C, Bb=B)
    return out_t.T


# speedup vs baseline: 1.0015x; 1.0015x over previous
import functools

import jax
import jax.numpy as jnp
from jax import lax
from jax.experimental import pallas as pl
from jax.experimental.pallas import tpu as pltpu
from jax.experimental.pallas import tpu_sc as plsc

NC = 2
NS = 16
LANES = 16
NW = NC * NS
CP = 128
CLS_PER_W = CP // NW


@functools.partial(jax.jit, static_argnames=("A", "A_P"))
def _sc_pool(logits_t, idx_flat, w_flat, *, A, A_P):
    V, B = logits_t.shape
    n_ent = CP * A_P
    vec_b = B // LANES

    mesh = plsc.VectorSubcoreMesh(
        core_axis_name="c", subcore_axis_name="s", num_cores=NC, num_subcores=NS
    )

    @functools.partial(
        pl.kernel,
        mesh=mesh,
        compiler_params=pltpu.CompilerParams(needs_layout_passes=False),
        out_type=jax.ShapeDtypeStruct((CP, B), jnp.float32),
        scratch_types=[
            pltpu.VMEM((CP, A_P), jnp.int32),
            pltpu.VMEM((n_ent,), jnp.float32),
            pltpu.VMEM((A * LANES,), jnp.float32),
            pltpu.VMEM((A_P, B), jnp.float32),
            pltpu.VMEM((B,), jnp.float32),
            pltpu.SemaphoreType.DMA,
        ],
    )
    def pool(lt_hbm, idx_hbm, w_hbm, out_hbm, idx_v, w_v, ws_v, rows_v, acc_v,
             sem):
        wid = lax.axis_index("s") * NC + lax.axis_index("c")
        pltpu.sync_copy(idx_hbm, idx_v)
        pltpu.sync_copy(w_hbm, w_v)

        def cls_body(k, carry):
            c = wid * CLS_PER_W + k
            ent0 = c * A_P
            pltpu.async_copy(lt_hbm.at[idx_v.at[c]], rows_v, sem).wait()

            def splat(a, c2):
                ws_v[pl.ds(a * LANES, LANES)] = plsc.load_gather(
                    w_v, [jnp.full((LANES,), ent0 + a, jnp.int32)]
                )
                return c2

            lax.fori_loop(0, A, splat, 0)

            def col_body(q, c2):
                sl = pl.ds(q * LANES, LANES)

                def a_body(a, acc):
                    return acc + rows_v[a, sl] * ws_v[pl.ds(a * LANES, LANES)]

                acc_v[sl] = lax.fori_loop(
                    0, A, a_body, jnp.zeros((LANES,), jnp.float32)
                )
                return c2

            lax.fori_loop(0, vec_b, col_body, 0)
            pltpu.sync_copy(acc_v, out_hbm.at[c])
            return carry

        lax.fori_loop(0, CLS_PER_W, cls_body, 0)

    return pool(logits_t, idx_flat, w_flat)


def _lse_body(x_ref, out_ref, m_ref, sum_ref, *, nv, vb):
    j = pl.program_id(0)
    i = pl.program_id(1)

    LOG2E = 1.4426950408889634
    x = x_ref[...]
    bm = jnp.max(x, axis=0, keepdims=True)
    ones = jnp.ones((1, vb), jnp.float32)

    @pl.when(j == 0)
    def _init():
        m_ref[i] = bm
        e = jnp.exp2(x * LOG2E - bm * LOG2E)
        sum_ref[i] = jnp.dot(ones, e, preferred_element_type=jnp.float32)

    @pl.when(j > 0)
    def _accum():
        m_old = m_ref[i]
        m_new = jnp.maximum(m_old, bm)
        alpha = jnp.where(m_old == m_new, 1.0, jnp.exp(m_old - m_new))
        e = jnp.exp2(x * LOG2E - m_new * LOG2E)
        es = jnp.dot(ones, e, preferred_element_type=jnp.float32)
        sum_ref[i] = sum_ref[i] * alpha + es
        m_ref[i] = m_new

    @pl.when(j == nv - 1)
    def _finalize():
        out_ref[...] = m_ref[i] + jnp.log(sum_ref[i])


@functools.partial(jax.jit, static_argnames=("Bb", "vb"))
def _lse(logits_t, *, Bb, vb):
    V, B = logits_t.shape
    nb = B // Bb
    nv = V // vb

    return pl.pallas_call(
        functools.partial(_lse_body, nv=nv, vb=vb),
        grid=(nv, nb),
        in_specs=[pl.BlockSpec((vb, Bb), lambda j, i: (j, i))],
        out_specs=pl.BlockSpec((1, Bb), lambda j, i: (0, i)),
        out_shape=jax.ShapeDtypeStruct((1, B), jnp.float32),
        scratch_shapes=[
            pltpu.VMEM((nb, 1, Bb), jnp.float32),
            pltpu.VMEM((nb, 1, Bb), jnp.float32),
        ],
        compiler_params=pltpu.CompilerParams(
            dimension_semantics=("arbitrary", "arbitrary"),
        ),
    )(logits_t)


def _combine_body(pooled_ref, norm_ref, mask_ref, out_ref, *, C):
    mask = mask_ref[...]
    summask = jnp.sum(mask, axis=1, keepdims=True)
    denom = jnp.clip(summask, 1e-9, None)
    inv = 1.0 / denom
    scale = summask * inv
    res = pooled_ref[...] * inv - scale * norm_ref[...]
    out_ref[...] = res[:C, :]


@functools.partial(jax.jit, static_argnames=("C", "Bb"))
def _combine(pooled, norm, mask_cp, *, C, Bb):
    _, B = pooled.shape
    nb = B // Bb

    return pl.pallas_call(
        functools.partial(_combine_body, C=C),
        grid=(nb,),
        in_specs=[
            pl.BlockSpec((CP, Bb), lambda i: (0, i)),
            pl.BlockSpec((1, Bb), lambda i: (0, i)),
            pl.BlockSpec(mask_cp.shape, lambda i: (0, 0)),
        ],
        out_specs=pl.BlockSpec((C, Bb), lambda i: (0, i)),
        out_shape=jax.ShapeDtypeStruct((C, B), jnp.float32),
    )(pooled, norm, mask_cp)


def kernel(logits, word2label, label_words_mask):
    B, V = logits.shape
    C, A = word2label.shape
    assert C <= CP and B % (LANES * NW) == 0

    A_P = -(-A // 8) * 8
    idx_cp = jnp.zeros((CP, A_P), jnp.int32).at[:C, :A].set(word2label)
    w_cp = jnp.zeros((CP, A_P), jnp.float32).at[:C, :A].set(
        label_words_mask.astype(jnp.float32)
    )
    mask_cp = w_cp[:, :A]

    lt = logits.T
    pooled = _sc_pool(lt, idx_cp, w_cp.reshape(-1), A=A, A_P=A_P)

    vb = next(d for d in (4000, 2000, 1000, 800, 500, 400, 250, 200, 100, 50,
                          25, 20, 10, 8, 5, 4, 2, 1) if V % d == 0)
    norm = _lse(lt, Bb=B, vb=vb)

    out_t = _combine(pooled, norm, mask_cp, C=C, Bb=B)
    return out_t.T
